# speedup vs baseline: 6.3209x; 6.3209x over previous
import functools

import jax
import jax.numpy as jnp
from jax import lax
from jax.experimental import pallas as pl
from jax.experimental.pallas import tpu as pltpu
from jax.experimental.pallas import tpu_sc as plsc

_NC = 2
_NS = 16
_NW = _NC * _NS
_CHUNK = 128


@functools.lru_cache(maxsize=None)
def _make_lookup(B, D):
    b_per_w = B // _NW
    n_chunks = b_per_w // _CHUNK
    mesh = plsc.VectorSubcoreMesh(core_axis_name="c", subcore_axis_name="s")

    @functools.partial(
        pl.kernel,
        mesh=mesh,
        out_type=jax.ShapeDtypeStruct((B, D), jnp.float32),
        scratch_types=[
            pltpu.VMEM((b_per_w,), jnp.int32),
            pltpu.VMEM((_CHUNK, D), jnp.float32),
            pltpu.SemaphoreType.DMA,
        ],
    )
    def lookup(table_hbm, idx_hbm, out_hbm, idx_v, rows_v, sem):
        wid = lax.axis_index("s") * _NC + lax.axis_index("c")
        base = wid * b_per_w
        pltpu.sync_copy(idx_hbm.at[pl.ds(base, b_per_w)], idx_v)

        def body(i, carry):
            off = pl.multiple_of(i * _CHUNK, _CHUNK)
            pltpu.async_copy(
                table_hbm.at[idx_v.at[pl.ds(off, _CHUNK)]], rows_v, sem
            ).wait()
            pltpu.sync_copy(rows_v, out_hbm.at[pl.ds(base + off, _CHUNK)])
            return carry

        lax.fori_loop(0, n_chunks, body, 0)

    return lookup


def kernel(input_batch, table):
    bsz, seq = input_batch.shape
    _, d = table.shape
    idx = input_batch.reshape(-1).astype(jnp.int32)
    out = _make_lookup(bsz * seq, d)(table, idx)
    return out.reshape(bsz, seq, d)


# speedup vs baseline: 7.4872x; 1.1845x over previous
import functools

import jax
import jax.numpy as jnp
from jax import lax
from jax.experimental import pallas as pl
from jax.experimental.pallas import tpu as pltpu
from jax.experimental.pallas import tpu_sc as plsc

_NC = 2
_NS = 16
_NW = _NC * _NS
_CHUNK = 128


@functools.lru_cache(maxsize=None)
def _make_lookup(B, D):
    b_per_w = B // _NW
    n_chunks = b_per_w // _CHUNK
    mesh = plsc.VectorSubcoreMesh(core_axis_name="c", subcore_axis_name="s")

    @functools.partial(
        pl.kernel,
        mesh=mesh,
        out_type=jax.ShapeDtypeStruct((B, D), jnp.float32),
        scratch_types=[
            pltpu.VMEM((b_per_w,), jnp.int32),
            pltpu.VMEM((2 * _CHUNK, D), jnp.float32),
            pltpu.SemaphoreType.DMA,
            pltpu.SemaphoreType.DMA,
        ],
    )
    def lookup(table_hbm, idx_hbm, out_hbm, idx_v, rows_v, gsem, ssem):
        wid = lax.axis_index("s") * _NC + lax.axis_index("c")
        base = wid * b_per_w
        pltpu.sync_copy(idx_hbm.at[pl.ds(base, b_per_w)], idx_v)

        def issue_gather(g, slot):
            off = pl.multiple_of(g * _CHUNK, _CHUNK)
            pltpu.async_copy(
                table_hbm.at[idx_v.at[pl.ds(off, _CHUNK)]],
                rows_v.at[pl.ds(slot * _CHUNK, _CHUNK)],
                gsem,
            )

        def wait_one(sem):
            pltpu.make_async_copy(
                out_hbm.at[pl.ds(base, _CHUNK)],
                rows_v.at[pl.ds(0, _CHUNK)],
                sem,
            ).wait()

        issue_gather(0, 0)

        def body(g, carry):
            slot = lax.rem(g, 2)
            wait_one(gsem)
            pltpu.async_copy(
                rows_v.at[pl.ds(slot * _CHUNK, _CHUNK)],
                out_hbm.at[pl.ds(base + pl.multiple_of(g * _CHUNK, _CHUNK), _CHUNK)],
                ssem,
            )

            @pl.when(g + 1 < n_chunks)
            def _():
                @pl.when(g >= 1)
                def _():
                    wait_one(ssem)
                issue_gather(g + 1, 1 - slot)

            return carry

        lax.fori_loop(0, n_chunks, body, 0)
        wait_one(ssem)
        wait_one(ssem)

    return lookup


def kernel(input_batch, table):
    bsz, seq = input_batch.shape
    _, d = table.shape
    idx = input_batch.reshape(-1).astype(jnp.int32)
    out = _make_lookup(bsz * seq, d)(table, idx)
    return out.reshape(bsz, seq, d)


# speedup vs baseline: 9.3288x; 1.2460x over previous
import functools

import jax
import jax.numpy as jnp
from jax import lax
from jax.experimental import pallas as pl
from jax.experimental.pallas import tpu as pltpu
from jax.experimental.pallas import tpu_sc as plsc

_NC = 2
_NS = 16
_NW = _NC * _NS
_CHUNK = 128


@functools.lru_cache(maxsize=None)
def _make_lookup(B, D):
    b_per_w = B // _NW
    n_chunks = b_per_w // _CHUNK
    mesh = plsc.VectorSubcoreMesh(core_axis_name="c", subcore_axis_name="s")

    nbuf = 6
    gdepth = 3

    @functools.partial(
        pl.kernel,
        mesh=mesh,
        out_type=jax.ShapeDtypeStruct((B, D), jnp.float32),
        scratch_types=[
            pltpu.VMEM((b_per_w,), jnp.int32),
            pltpu.VMEM((nbuf * _CHUNK, D), jnp.float32),
            pltpu.SemaphoreType.DMA,
            pltpu.SemaphoreType.DMA,
        ],
    )
    def lookup(table_hbm, idx_hbm, out_hbm, idx_v, rows_v, gsem, ssem):
        wid = lax.axis_index("s") * _NC + lax.axis_index("c")
        base = wid * b_per_w
        pltpu.sync_copy(idx_hbm.at[pl.ds(base, b_per_w)], idx_v)

        def issue_gather(g, slot):
            off = pl.multiple_of(g * _CHUNK, _CHUNK)
            pltpu.async_copy(
                table_hbm.at[idx_v.at[pl.ds(off, _CHUNK)]],
                rows_v.at[pl.ds(slot * _CHUNK, _CHUNK)],
                gsem,
            )

        def wait_one(sem):
            pltpu.make_async_copy(
                out_hbm.at[pl.ds(base, _CHUNK)],
                rows_v.at[pl.ds(0, _CHUNK)],
                sem,
            ).wait()

        for h in range(gdepth):
            issue_gather(h, h)

        def body(g, carry):
            slot = lax.rem(g, nbuf)
            wait_one(gsem)
            pltpu.async_copy(
                rows_v.at[pl.ds(slot * _CHUNK, _CHUNK)],
                out_hbm.at[pl.ds(base + pl.multiple_of(g * _CHUNK, _CHUNK), _CHUNK)],
                ssem,
            )
            nxt = g + gdepth

            @pl.when(nxt < n_chunks)
            def _():
                @pl.when(nxt >= nbuf)
                def _():
                    wait_one(ssem)
                issue_gather(nxt, lax.rem(nxt, nbuf))

            return carry

        lax.fori_loop(0, n_chunks, body, 0)
        for _ in range(nbuf):
            wait_one(ssem)

    return lookup


def kernel(input_batch, table):
    bsz, seq = input_batch.shape
    _, d = table.shape
    idx = input_batch.reshape(-1).astype(jnp.int32)
    out = _make_lookup(bsz * seq, d)(table, idx)
    return out.reshape(bsz, seq, d)
